# baseline (speedup 1.0000x reference)
import functools

import jax
import jax.numpy as jnp
from jax import lax
from jax.experimental import pallas as pl
from jax.experimental.pallas import tpu as pltpu
from jax.experimental.pallas import tpu_sc as plsc

VOCAB = 1000000
DIM = 64
BATCH = 4096
SEQ = 200
N = BATCH * SEQ

NUM_CORES = 2
NUM_SUBCORES = 16
NW = NUM_CORES * NUM_SUBCORES
PER_W = N // NW
CHUNK = 512
NCHUNK = PER_W // CHUNK

_mesh = plsc.VectorSubcoreMesh(core_axis_name="c", subcore_axis_name="s")


@functools.partial(
    pl.kernel,
    out_type=jax.ShapeDtypeStruct((N, DIM), jnp.float32),
    mesh=_mesh,
    scratch_types=[
        pltpu.VMEM((CHUNK,), jnp.int32),
        pltpu.VMEM((CHUNK,), jnp.int32),
        pltpu.VMEM((CHUNK, DIM), jnp.float32),
        pltpu.VMEM((CHUNK, DIM), jnp.float32),
        pltpu.SemaphoreType.DMA,
        pltpu.SemaphoreType.DMA,
    ],
    compiler_params=pltpu.CompilerParams(use_tc_tiling_on_sc=False),
)
def _embed_sc(ids_hbm, table_hbm, out_hbm,
              idx0, idx1, rows0, rows1, sem0, sem1):
    wid = lax.axis_index("s") * NUM_CORES + lax.axis_index("c")
    base = wid * PER_W

    bufs = ((idx0, rows0, sem0), (idx1, rows1, sem1))

    pltpu.sync_copy(ids_hbm.at[pl.ds(base, CHUNK)], idx0)
    pltpu.async_copy(table_hbm.at[idx0], rows0, sem0)

    def step(g):
        for b in range(2):
            idx_c, rows_c, sem_c = bufs[b]
            idx_n, rows_n, sem_n = bufs[1 - b]
            cur = g + b
            nxt = cur + 1

            @pl.when(nxt < NCHUNK)
            def _():
                pltpu.sync_copy(ids_hbm.at[pl.ds(base + nxt * CHUNK, CHUNK)],
                                idx_n)
                pltpu.async_copy(table_hbm.at[idx_n], rows_n, sem_n)

            pltpu.make_async_copy(table_hbm.at[idx_c], rows_c, sem_c).wait()
            pltpu.sync_copy(rows_c, out_hbm.at[pl.ds(base + cur * CHUNK, CHUNK)])

    pl.loop(0, NCHUNK, step=2)(step)


def kernel(token_ids, weights):
    flat = token_ids.reshape(N)
    out = _embed_sc(flat, weights)
    return out.reshape(BATCH, SEQ, DIM)


# speedup vs baseline: 1.0101x; 1.0101x over previous
import functools

import jax
import jax.numpy as jnp
from jax import lax
from jax.experimental import pallas as pl
from jax.experimental.pallas import tpu as pltpu
from jax.experimental.pallas import tpu_sc as plsc

VOCAB = 1000000
DIM = 64
BATCH = 4096
SEQ = 200
N = BATCH * SEQ

NUM_CORES = 2
NUM_SUBCORES = 16
NW = NUM_CORES * NUM_SUBCORES
PER_W = N // NW
CHUNK = 400
NCHUNK = PER_W // CHUNK
NBUF = 4

_mesh = plsc.VectorSubcoreMesh(core_axis_name="c", subcore_axis_name="s")


@functools.partial(
    pl.kernel,
    out_type=jax.ShapeDtypeStruct((N, DIM), jnp.float32),
    mesh=_mesh,
    scratch_types=[
        pltpu.VMEM((NCHUNK, CHUNK), jnp.int32),
        [pltpu.VMEM((CHUNK, DIM), jnp.float32)] * NBUF,
        [pltpu.SemaphoreType.DMA] * NBUF,
        [pltpu.SemaphoreType.DMA] * NBUF,
    ],
    compiler_params=pltpu.CompilerParams(use_tc_tiling_on_sc=False),
)
def _embed_sc(ids_hbm, table_hbm, out_hbm, idx_slab, rows, gsem, wsem):
    wid = lax.axis_index("s") * NUM_CORES + lax.axis_index("c")
    base = wid * PER_W

    pltpu.sync_copy(ids_hbm.at[wid], idx_slab)

    for i in range(NBUF - 1):
        pltpu.async_copy(table_hbm.at[idx_slab.at[i]], rows[i], gsem[i])

    def step(cur0):
        for b in range(NBUF):
            cur = cur0 + b
            pltpu.make_async_copy(table_hbm.at[idx_slab.at[0]],
                                  rows[b], gsem[b]).wait()
            pltpu.async_copy(rows[b],
                             out_hbm.at[pl.ds(base + cur * CHUNK, CHUNK)],
                             wsem[b])
            nb = (b - 1) % NBUF
            g = cur + NBUF - 1

            @pl.when(g < NCHUNK)
            def _():
                @pl.when(cur >= 1)
                def _():
                    pltpu.make_async_copy(
                        rows[nb], out_hbm.at[pl.ds(base, CHUNK)],
                        wsem[nb]).wait()

                pltpu.async_copy(table_hbm.at[idx_slab.at[g]],
                                 rows[nb], gsem[nb])

    pl.loop(0, NCHUNK, step=NBUF)(step)

    for b in range(NBUF):
        pltpu.make_async_copy(rows[b], out_hbm.at[pl.ds(base, CHUNK)],
                              wsem[b]).wait()


def kernel(token_ids, weights):
    ids = token_ids.reshape(NW, NCHUNK, CHUNK)
    out = _embed_sc(ids, weights)
    return out.reshape(BATCH, SEQ, DIM)
